# Initial kernel scaffold; baseline (speedup 1.0000x reference)
import jax
import jax.numpy as jnp
from jax.experimental import pallas as pl


def kernel(x, router_w, router_bias, shared_gate_w, shared_up_w, shared_down_w, gate_ws, up_ws, down_ws):
    raise NotImplementedError("write your pallas kernel here")


# speedup vs baseline: 1.5080x; 1.5080x over previous
import jax
import jax.numpy as jnp
from jax.experimental import pallas as pl
from jax.experimental.pallas import tpu as pltpu

D_MODEL = 1024
INTER = 512
NUM_E = 8
TOKEN_BLOCK = 1024


def _nt(a, b):
    return jax.lax.dot_general(
        a, b, (((1,), (1,)), ((), ())), preferred_element_type=jnp.float32
    )


def _moe_body(x_ref, rw_ref, rb_ref, sg_ref, su_ref, sd_ref,
              gw_ref, uw_ref, dw_ref, out_ref, wscr_ref):
    e = pl.program_id(1)
    xf = x_ref[...]
    xb = xf.astype(jnp.bfloat16)
    ne = rw_ref.shape[0]

    @pl.when(e == 0)
    def _init():
        logits = _nt(xb, rw_ref[...].astype(jnp.bfloat16)) + rb_ref[...]
        iota = jax.lax.broadcasted_iota(jnp.int32, logits.shape, 1)
        m0 = jnp.max(logits, axis=1, keepdims=True)
        i0 = jnp.min(jnp.where(logits == m0, iota, ne), axis=1, keepdims=True)
        oh0 = iota == i0
        masked = jnp.where(oh0, -jnp.inf, logits)
        m1 = jnp.max(masked, axis=1, keepdims=True)
        i1 = jnp.min(jnp.where(masked == m1, iota, ne), axis=1, keepdims=True)
        oh1 = iota == i1
        t = jnp.exp(m1 - m0)
        w0 = 1.0 / (1.0 + t)
        w1 = t / (1.0 + t)
        wscr_ref[...] = jnp.where(oh0, w0, 0.0) + jnp.where(oh1, w1, 0.0)

        g = _nt(xb, sg_ref[...].astype(jnp.bfloat16))
        u = _nt(xb, su_ref[...].astype(jnp.bfloat16))
        h = ((g * jax.nn.sigmoid(g)) * u).astype(jnp.bfloat16)
        out_ref[...] = _nt(h, sd_ref[...].astype(jnp.bfloat16))

    lane = jax.lax.broadcasted_iota(jnp.int32, wscr_ref.shape, 1)
    w_e = jnp.sum(jnp.where(lane == e, wscr_ref[...], 0.0), axis=1,
                  keepdims=True)
    g = _nt(xb, gw_ref[0].astype(jnp.bfloat16))
    u = _nt(xb, uw_ref[0].astype(jnp.bfloat16))
    h = (((g * jax.nn.sigmoid(g)) * u) * w_e).astype(jnp.bfloat16)
    out_ref[...] += _nt(h, dw_ref[0].astype(jnp.bfloat16))


def kernel(x, router_w, router_bias, shared_gate_w, shared_up_w,
           shared_down_w, gate_ws, up_ws, down_ws):
    b, s, d = x.shape
    n = b * s
    flat = x.reshape(n, d)
    rb = router_bias.reshape(1, NUM_E)
    grid = (n // TOKEN_BLOCK, NUM_E)
    out = pl.pallas_call(
        _moe_body,
        grid=grid,
        in_specs=[
            pl.BlockSpec((TOKEN_BLOCK, d), lambda t, e: (t, 0)),
            pl.BlockSpec((NUM_E, d), lambda t, e: (0, 0)),
            pl.BlockSpec((1, NUM_E), lambda t, e: (0, 0)),
            pl.BlockSpec((INTER, d), lambda t, e: (0, 0)),
            pl.BlockSpec((INTER, d), lambda t, e: (0, 0)),
            pl.BlockSpec((d, INTER), lambda t, e: (0, 0)),
            pl.BlockSpec((1, INTER, d), lambda t, e: (e, 0, 0)),
            pl.BlockSpec((1, INTER, d), lambda t, e: (e, 0, 0)),
            pl.BlockSpec((1, d, INTER), lambda t, e: (e, 0, 0)),
        ],
        out_specs=pl.BlockSpec((TOKEN_BLOCK, d), lambda t, e: (t, 0)),
        out_shape=jax.ShapeDtypeStruct((n, d), jnp.float32),
        scratch_shapes=[pltpu.VMEM((TOKEN_BLOCK, NUM_E), jnp.float32)],
        compiler_params=pltpu.CompilerParams(
            dimension_semantics=("parallel", "arbitrary")),
    )(flat, router_w, rb, shared_gate_w, shared_up_w, shared_down_w,
      gate_ws, up_ws, down_ws)
    return out.reshape(b, s, d)
